# baseline (speedup 1.0000x reference)
import functools

import jax
import jax.numpy as jnp
from jax import lax
from jax.experimental import pallas as pl
from jax.experimental.pallas import tpu as pltpu
from jax.experimental.pallas import tpu_sc as plsc

N_NODES = 10000
N_EDGES = 320000
D = 128

NC = 2
NS = 16
NW = NC * NS

CH = 128
CPW = 80
NCH = NW * CPW
EPAD = NCH * CH

NP = 10240
RPT = NP // NS
DW = 16

_mesh = plsc.VectorSubcoreMesh(core_axis_name="c", subcore_axis_name="s")


def _zero_rows(ref, nrows, width):
    groups = width // 16

    def body(i, carry):
        for j in range(groups):
            ref[i, pl.ds(j * 16, 16)] = jnp.zeros((16,), jnp.float32)
        return carry

    lax.fori_loop(0, nrows, body, 0)


def _fill_1d(ref, n, value):
    def body(i, carry):
        ref[pl.ds(i * 16, 16)] = jnp.full((16,), value, jnp.float32)
        return carry

    lax.fori_loop(0, n // 16, body, 0)


@functools.partial(
    pl.kernel,
    out_type=jax.ShapeDtypeStruct((NC, 2, NP), jnp.float32),
    mesh=_mesh,
    scratch_types=[
        pltpu.VMEM((CPW, CH), jnp.int32),
        pltpu.VMEM((CPW, CH), jnp.int32),
        pltpu.VMEM((CH,), jnp.float32),
        pltpu.VMEM((RPT,), jnp.float32),
        pltpu.VMEM_SHARED((NP,), jnp.float32),
        pltpu.VMEM_SHARED((NP,), jnp.float32),
    ],
)
def _sc_degrees(src_hbm, dst_hbm, out_hbm, sidx, didx, ones_v, zb, acc_s, acc_d):
    c = lax.axis_index("c")
    s = lax.axis_index("s")
    wid = c * NS + s

    _fill_1d(zb, RPT, 0.0)
    _fill_1d(ones_v, CH, 1.0)

    pltpu.sync_copy(zb, acc_s.at[pl.ds(s * RPT, RPT)])
    pltpu.sync_copy(zb, acc_d.at[pl.ds(s * RPT, RPT)])
    plsc.subcore_barrier()

    base = wid * CPW
    pltpu.sync_copy(src_hbm.at[pl.ds(base, CPW)], sidx)
    pltpu.sync_copy(dst_hbm.at[pl.ds(base, CPW)], didx)

    def body(k, carry):
        pltpu.sync_copy(ones_v, acc_s.at[sidx.at[k]], add=True)
        pltpu.sync_copy(ones_v, acc_d.at[didx.at[k]], add=True)
        return carry

    lax.fori_loop(0, CPW, body, 0)
    plsc.subcore_barrier()

    pltpu.sync_copy(acc_s.at[pl.ds(s * RPT, RPT)], out_hbm.at[c, 0, pl.ds(s * RPT, RPT)])
    pltpu.sync_copy(acc_d.at[pl.ds(s * RPT, RPT)], out_hbm.at[c, 1, pl.ds(s * RPT, RPT)])


GC = 16
NG = CPW // GC


@functools.partial(
    pl.kernel,
    out_type=jax.ShapeDtypeStruct((NC, NP, D), jnp.float32),
    mesh=_mesh,
    scratch_types=[
        pltpu.VMEM((GC, CH), jnp.int32),
        pltpu.VMEM((GC, CH), jnp.int32),
        pltpu.VMEM((CH, D), jnp.float32),
        pltpu.VMEM((CH, D), jnp.float32),
        pltpu.SemaphoreType.DMA,
        pltpu.SemaphoreType.DMA,
        pltpu.SemaphoreType.DMA,
        pltpu.SemaphoreType.DMA,
        pltpu.VMEM_SHARED((NP, D), jnp.float32),
    ],
)
def _sc_msgpass(h_hbm, src_hbm, dst_hbm, out_hbm, sidx, didx, rows0, rows1,
                sem0, sem1, tem0, tem1, acc):
    c = lax.axis_index("c")
    s = lax.axis_index("s")
    wid = c * NS + s

    _zero_rows(rows0, CH, D)

    for t in range(RPT // CH):
        pltpu.sync_copy(rows0, acc.at[pl.ds(s * RPT + t * CH, CH)])
    plsc.subcore_barrier()

    base = wid * CPW

    def group(g, carry):
        pltpu.sync_copy(src_hbm.at[pl.ds(base + g * GC, GC)], sidx)
        pltpu.sync_copy(dst_hbm.at[pl.ds(base + g * GC, GC)], didx)
        pltpu.async_copy(h_hbm.at[sidx.at[0]], rows0, sem0)
        pltpu.async_copy(h_hbm.at[sidx.at[1]], rows1, sem1)

        def body(j, carry2):
            k = 2 * j
            pltpu.make_async_copy(h_hbm.at[sidx.at[k]], rows0, sem0).wait()
            pltpu.async_copy(rows0, acc.at[didx.at[k]], tem0, add=True)
            pltpu.make_async_copy(h_hbm.at[sidx.at[k + 1]], rows1, sem1).wait()
            pltpu.async_copy(rows1, acc.at[didx.at[k + 1]], tem1, add=True)
            pltpu.make_async_copy(rows0, acc.at[didx.at[k]], tem0).wait()
            pltpu.async_copy(h_hbm.at[sidx.at[k + 2]], rows0, sem0)
            pltpu.make_async_copy(rows1, acc.at[didx.at[k + 1]], tem1).wait()
            pltpu.async_copy(h_hbm.at[sidx.at[k + 3]], rows1, sem1)
            return carry2

        lax.fori_loop(0, GC // 2 - 1, body, 0)
        kl = GC - 2
        pltpu.make_async_copy(h_hbm.at[sidx.at[kl]], rows0, sem0).wait()
        pltpu.async_copy(rows0, acc.at[didx.at[kl]], tem0, add=True)
        pltpu.make_async_copy(h_hbm.at[sidx.at[kl + 1]], rows1, sem1).wait()
        pltpu.async_copy(rows1, acc.at[didx.at[kl + 1]], tem1, add=True)
        pltpu.make_async_copy(rows0, acc.at[didx.at[kl]], tem0).wait()
        pltpu.make_async_copy(rows1, acc.at[didx.at[kl + 1]], tem1).wait()
        return carry

    lax.fori_loop(0, NG, group, 0)
    plsc.subcore_barrier()

    for t in range(RPT // CH):
        r = s * RPT + t * CH
        pltpu.sync_copy(acc.at[pl.ds(r, CH)], out_hbm.at[c, pl.ds(r, CH)])


def _tc_pre_body(x_ref, w_ref, dsp_ref, ddp_ref, h1_ref, ns_ref, nd_ref):
    ds = (dsp_ref[0] + dsp_ref[1])[:N_NODES]
    dd = (ddp_ref[0] + ddp_ref[1])[:N_NODES]
    ns = lax.rsqrt(jnp.maximum(ds, 1.0))
    nd = lax.rsqrt(jnp.maximum(dd, 1.0))
    u = jnp.dot(x_ref[...], w_ref[...], preferred_element_type=jnp.float32)
    h1_ref[...] = u * ns
    ns_ref[...] = ns
    nd_ref[...] = nd


def _tc_mid_body(p_ref, ns_ref, nd_ref, b1_ref, w2_ref, h2_ref):
    agg = p_ref[0, :N_NODES, :] + p_ref[1, :N_NODES, :]
    h = jnp.maximum(agg * nd_ref[...] + b1_ref[...][None, :], 0.0)
    h2_ref[...] = jnp.dot(h, w2_ref[...], preferred_element_type=jnp.float32) * ns_ref[...]


def _tc_post_body(p_ref, nd_ref, b2_ref, out_ref):
    agg = p_ref[0, :N_NODES, :] + p_ref[1, :N_NODES, :]
    out_ref[...] = agg * nd_ref[...] + b2_ref[...][None, :]


def kernel(in_feat, edge_index, W1, b1, W2, b2):
    src = edge_index[0]
    dst = edge_index[1]
    npad = EPAD - N_EDGES
    ar = jnp.arange(npad, dtype=jnp.int32)
    pad_g = (ar * 97) % N_NODES
    pad_d = N_NODES + (ar % (NP - N_NODES))
    src_g = jnp.concatenate([src, pad_g]).reshape(NCH, CH)
    src_d = jnp.concatenate([src, pad_d]).reshape(NCH, CH)
    dst_d = jnp.concatenate([dst, pad_d]).reshape(NCH, CH)

    degs = _sc_degrees(src_d, dst_d)
    dsp = degs[:, 0, :, None]
    ddp = degs[:, 1, :, None]

    h1, ns, nd = pl.pallas_call(
        _tc_pre_body,
        out_shape=[
            jax.ShapeDtypeStruct((N_NODES, D), jnp.float32),
            jax.ShapeDtypeStruct((N_NODES, 1), jnp.float32),
            jax.ShapeDtypeStruct((N_NODES, 1), jnp.float32),
        ],
    )(in_feat, W1, dsp, ddp)

    p1 = _sc_msgpass(h1, src_g, dst_d)

    h2 = pl.pallas_call(
        _tc_mid_body,
        out_shape=jax.ShapeDtypeStruct((N_NODES, D), jnp.float32),
    )(p1, ns, nd, b1, W2)

    p2 = _sc_msgpass(h2, src_g, dst_d)

    out = pl.pallas_call(
        _tc_post_body,
        out_shape=jax.ShapeDtypeStruct((N_NODES, D), jnp.float32),
    )(p2, nd, b2)
    return out


# speedup vs baseline: 1.1014x; 1.1014x over previous
import functools

import jax
import jax.numpy as jnp
from jax import lax
from jax.experimental import pallas as pl
from jax.experimental.pallas import tpu as pltpu
from jax.experimental.pallas import tpu_sc as plsc

N_NODES = 10000
N_EDGES = 320000
D = 128

NC = 2
NS = 16
NW = NC * NS

CH = 128
CPW = 80
NCH = NW * CPW
EPAD = NCH * CH

NP = 10240
RPT = NP // NS
DW = 16

_mesh = plsc.VectorSubcoreMesh(core_axis_name="c", subcore_axis_name="s")


def _zero_rows(ref, nrows, width):
    groups = width // 16

    def body(i, carry):
        for j in range(groups):
            ref[i, pl.ds(j * 16, 16)] = jnp.zeros((16,), jnp.float32)
        return carry

    lax.fori_loop(0, nrows, body, 0)


def _fill_1d(ref, n, value):
    def body(i, carry):
        ref[pl.ds(i * 16, 16)] = jnp.full((16,), value, jnp.float32)
        return carry

    lax.fori_loop(0, n // 16, body, 0)


@functools.partial(
    pl.kernel,
    out_type=jax.ShapeDtypeStruct((NC, 2, NP), jnp.float32),
    mesh=_mesh,
    scratch_types=[
        pltpu.VMEM((CPW, CH), jnp.int32),
        pltpu.VMEM((CPW, CH), jnp.int32),
        pltpu.VMEM((CH,), jnp.float32),
        pltpu.VMEM((RPT,), jnp.float32),
        pltpu.SemaphoreType.DMA,
        pltpu.SemaphoreType.DMA,
        pltpu.VMEM_SHARED((NP,), jnp.float32),
        pltpu.VMEM_SHARED((NP,), jnp.float32),
    ],
)
def _sc_degrees(src_hbm, dst_hbm, out_hbm, sidx, didx, ones_v, zb, sa, sb,
                acc_s, acc_d):
    c = lax.axis_index("c")
    s = lax.axis_index("s")
    wid = c * NS + s

    _fill_1d(zb, RPT, 0.0)
    _fill_1d(ones_v, CH, 1.0)

    pltpu.sync_copy(zb, acc_s.at[pl.ds(s * RPT, RPT)])
    pltpu.sync_copy(zb, acc_d.at[pl.ds(s * RPT, RPT)])
    plsc.subcore_barrier()

    base = wid * CPW
    pltpu.sync_copy(src_hbm.at[pl.ds(base, CPW)], sidx)
    pltpu.sync_copy(dst_hbm.at[pl.ds(base, CPW)], didx)

    pltpu.async_copy(ones_v, acc_s.at[sidx.at[0]], sa, add=True)
    pltpu.async_copy(ones_v, acc_d.at[didx.at[0]], sb, add=True)

    def body(k, carry):
        pltpu.async_copy(ones_v, acc_s.at[sidx.at[k]], sa, add=True)
        pltpu.async_copy(ones_v, acc_d.at[didx.at[k]], sb, add=True)
        pltpu.make_async_copy(ones_v, acc_s.at[sidx.at[0]], sa).wait()
        pltpu.make_async_copy(ones_v, acc_d.at[didx.at[0]], sb).wait()
        return carry

    lax.fori_loop(1, CPW, body, 0)
    pltpu.make_async_copy(ones_v, acc_s.at[sidx.at[0]], sa).wait()
    pltpu.make_async_copy(ones_v, acc_d.at[didx.at[0]], sb).wait()
    plsc.subcore_barrier()

    pltpu.sync_copy(acc_s.at[pl.ds(s * RPT, RPT)], out_hbm.at[c, 0, pl.ds(s * RPT, RPT)])
    pltpu.sync_copy(acc_d.at[pl.ds(s * RPT, RPT)], out_hbm.at[c, 1, pl.ds(s * RPT, RPT)])


GC = 16
NG = CPW // GC


@functools.partial(
    pl.kernel,
    out_type=jax.ShapeDtypeStruct((NC, NP, D), jnp.float32),
    mesh=_mesh,
    scratch_types=[
        pltpu.VMEM((GC, CH), jnp.int32),
        pltpu.VMEM((GC, CH), jnp.int32),
        pltpu.VMEM((CH, D), jnp.float32),
        pltpu.VMEM((CH, D), jnp.float32),
        pltpu.SemaphoreType.DMA,
        pltpu.SemaphoreType.DMA,
        pltpu.VMEM_SHARED((NP, D), jnp.float32),
    ],
)
def _sc_msgpass(h_hbm, src_hbm, dst_hbm, out_hbm, sidx, didx, rows0, rows1,
                sem0, sem1, acc):
    c = lax.axis_index("c")
    s = lax.axis_index("s")
    wid = c * NS + s

    _zero_rows(rows0, CH, D)

    for t in range(RPT // CH):
        pltpu.sync_copy(rows0, acc.at[pl.ds(s * RPT + t * CH, CH)])
    plsc.subcore_barrier()

    base = wid * CPW

    def group(g, carry):
        pltpu.sync_copy(src_hbm.at[pl.ds(base + g * GC, GC)], sidx)
        pltpu.sync_copy(dst_hbm.at[pl.ds(base + g * GC, GC)], didx)
        pltpu.async_copy(h_hbm.at[sidx.at[0]], rows0, sem0)

        def body(j, carry2):
            k = 2 * j
            pltpu.make_async_copy(h_hbm.at[sidx.at[k]], rows0, sem0).wait()
            pltpu.async_copy(h_hbm.at[sidx.at[k + 1]], rows1, sem1)
            pltpu.sync_copy(rows0, acc.at[didx.at[k]], add=True)
            pltpu.make_async_copy(h_hbm.at[sidx.at[k + 1]], rows1, sem1).wait()
            pltpu.async_copy(h_hbm.at[sidx.at[k + 2]], rows0, sem0)
            pltpu.sync_copy(rows1, acc.at[didx.at[k + 1]], add=True)
            return carry2

        lax.fori_loop(0, GC // 2 - 1, body, 0)
        kl = GC - 2
        pltpu.make_async_copy(h_hbm.at[sidx.at[kl]], rows0, sem0).wait()
        pltpu.async_copy(h_hbm.at[sidx.at[kl + 1]], rows1, sem1)
        pltpu.sync_copy(rows0, acc.at[didx.at[kl]], add=True)
        pltpu.make_async_copy(h_hbm.at[sidx.at[kl + 1]], rows1, sem1).wait()
        pltpu.sync_copy(rows1, acc.at[didx.at[kl + 1]], add=True)
        return carry

    lax.fori_loop(0, NG, group, 0)
    plsc.subcore_barrier()

    for t in range(RPT // CH):
        r = s * RPT + t * CH
        pltpu.sync_copy(acc.at[pl.ds(r, CH)], out_hbm.at[c, pl.ds(r, CH)])


def _tc_mm_body(x_ref, w_ref, u_ref):
    u_ref[...] = jnp.dot(x_ref[...], w_ref[...], preferred_element_type=jnp.float32)


def _tc_scale_body(u_ref, dsp_ref, ddp_ref, h1_ref, ns_ref, nd_ref):
    ds = (dsp_ref[0] + dsp_ref[1])[:N_NODES]
    dd = (ddp_ref[0] + ddp_ref[1])[:N_NODES]
    ns = lax.rsqrt(jnp.maximum(ds, 1.0))
    nd = lax.rsqrt(jnp.maximum(dd, 1.0))
    h1_ref[...] = u_ref[...] * ns
    ns_ref[...] = ns
    nd_ref[...] = nd


def _tc_mid_body(p_ref, ns_ref, nd_ref, b1_ref, w2_ref, h2_ref):
    agg = p_ref[0, :N_NODES, :] + p_ref[1, :N_NODES, :]
    h = jnp.maximum(agg * nd_ref[...] + b1_ref[...][None, :], 0.0)
    h2_ref[...] = jnp.dot(h, w2_ref[...], preferred_element_type=jnp.float32) * ns_ref[...]


def _tc_post_body(p_ref, nd_ref, b2_ref, out_ref):
    agg = p_ref[0, :N_NODES, :] + p_ref[1, :N_NODES, :]
    out_ref[...] = agg * nd_ref[...] + b2_ref[...][None, :]


def kernel(in_feat, edge_index, W1, b1, W2, b2):
    src = edge_index[0]
    dst = edge_index[1]
    npad = EPAD - N_EDGES
    ar = jnp.arange(npad, dtype=jnp.int32)
    pad_g = (ar * 97) % N_NODES
    pad_d = N_NODES + (ar % (NP - N_NODES))
    src_g = jnp.concatenate([src, pad_g]).reshape(NCH, CH)
    src_d = jnp.concatenate([src, pad_d]).reshape(NCH, CH)
    dst_d = jnp.concatenate([dst, pad_d]).reshape(NCH, CH)

    u1 = pl.pallas_call(
        _tc_mm_body,
        out_shape=jax.ShapeDtypeStruct((N_NODES, D), jnp.float32),
    )(in_feat, W1)

    degs = _sc_degrees(src_d, dst_d)
    dsp = degs[:, 0, :, None]
    ddp = degs[:, 1, :, None]

    h1, ns, nd = pl.pallas_call(
        _tc_scale_body,
        out_shape=[
            jax.ShapeDtypeStruct((N_NODES, D), jnp.float32),
            jax.ShapeDtypeStruct((N_NODES, 1), jnp.float32),
            jax.ShapeDtypeStruct((N_NODES, 1), jnp.float32),
        ],
    )(u1, dsp, ddp)

    p1 = _sc_msgpass(h1, src_g, dst_d)

    h2 = pl.pallas_call(
        _tc_mid_body,
        out_shape=jax.ShapeDtypeStruct((N_NODES, D), jnp.float32),
    )(p1, ns, nd, b1, W2)

    p2 = _sc_msgpass(h2, src_g, dst_d)

    out = pl.pallas_call(
        _tc_post_body,
        out_shape=jax.ShapeDtypeStruct((N_NODES, D), jnp.float32),
    )(p2, nd, b2)
    return out


# speedup vs baseline: 1.1823x; 1.0735x over previous
import functools

import jax
import jax.numpy as jnp
import numpy as np
from jax import lax
from jax.experimental import pallas as pl
from jax.experimental.pallas import tpu as pltpu
from jax.experimental.pallas import tpu_sc as plsc

N_NODES = 10000
N_EDGES = 320000
D = 128

NC = 2
NS = 16
NW = NC * NS

CH = 128
CPW = 80
NCH = NW * CPW
EPAD = NCH * CH

NP = 10240
RPT = NP // NS
DW = 16

_mesh = plsc.VectorSubcoreMesh(core_axis_name="c", subcore_axis_name="s")


def _zero_rows(ref, nrows, width):
    groups = width // 16

    def body(i, carry):
        for j in range(groups):
            ref[i, pl.ds(j * 16, 16)] = jnp.zeros((16,), jnp.float32)
        return carry

    lax.fori_loop(0, nrows, body, 0)


def _fill_1d(ref, n, value):
    def body(i, carry):
        ref[pl.ds(i * 16, 16)] = jnp.full((16,), value, jnp.float32)
        return carry

    lax.fori_loop(0, n // 16, body, 0)


@functools.partial(
    pl.kernel,
    out_type=jax.ShapeDtypeStruct((NC, 2, NP), jnp.float32),
    mesh=_mesh,
    scratch_types=[
        pltpu.VMEM((CPW, CH), jnp.int32),
        pltpu.VMEM((CPW, CH), jnp.int32),
        pltpu.VMEM((CH,), jnp.float32),
        pltpu.VMEM((RPT,), jnp.float32),
        pltpu.SemaphoreType.DMA,
        pltpu.SemaphoreType.DMA,
        pltpu.VMEM_SHARED((NP,), jnp.float32),
        pltpu.VMEM_SHARED((NP,), jnp.float32),
    ],
)
def _sc_degrees(src_hbm, dst_hbm, out_hbm, sidx, didx, ones_v, zb, sa, sb,
                acc_s, acc_d):
    c = lax.axis_index("c")
    s = lax.axis_index("s")
    wid = c * NS + s

    _fill_1d(zb, RPT, 0.0)
    _fill_1d(ones_v, CH, 1.0)

    pltpu.sync_copy(zb, acc_s.at[pl.ds(s * RPT, RPT)])
    pltpu.sync_copy(zb, acc_d.at[pl.ds(s * RPT, RPT)])
    plsc.subcore_barrier()

    base = wid * CPW
    pltpu.sync_copy(src_hbm.at[pl.ds(base, CPW)], sidx)
    pltpu.sync_copy(dst_hbm.at[pl.ds(base, CPW)], didx)

    pltpu.async_copy(ones_v, acc_s.at[sidx.at[0]], sa, add=True)
    pltpu.async_copy(ones_v, acc_d.at[didx.at[0]], sb, add=True)

    def body(k, carry):
        pltpu.async_copy(ones_v, acc_s.at[sidx.at[k]], sa, add=True)
        pltpu.async_copy(ones_v, acc_d.at[didx.at[k]], sb, add=True)
        pltpu.make_async_copy(ones_v, acc_s.at[sidx.at[0]], sa).wait()
        pltpu.make_async_copy(ones_v, acc_d.at[didx.at[0]], sb).wait()
        return carry

    lax.fori_loop(1, CPW, body, 0)
    pltpu.make_async_copy(ones_v, acc_s.at[sidx.at[0]], sa).wait()
    pltpu.make_async_copy(ones_v, acc_d.at[didx.at[0]], sb).wait()
    plsc.subcore_barrier()

    pltpu.sync_copy(acc_s.at[pl.ds(s * RPT, RPT)], out_hbm.at[c, 0, pl.ds(s * RPT, RPT)])
    pltpu.sync_copy(acc_d.at[pl.ds(s * RPT, RPT)], out_hbm.at[c, 1, pl.ds(s * RPT, RPT)])


GC = 16
NG = CPW // GC


@functools.partial(
    pl.kernel,
    out_type=jax.ShapeDtypeStruct((NC, NP, D), jnp.float32),
    mesh=_mesh,
    scratch_types=[
        pltpu.VMEM((GC, CH), jnp.int32),
        pltpu.VMEM((GC, CH), jnp.int32),
        pltpu.VMEM((CH, D), jnp.float32),
        pltpu.VMEM((CH, D), jnp.float32),
        pltpu.SemaphoreType.DMA,
        pltpu.SemaphoreType.DMA,
        pltpu.VMEM_SHARED((NP, D), jnp.float32),
    ],
)
def _sc_msgpass(h_hbm, src_hbm, dst_hbm, out_hbm, sidx, didx, rows0, rows1,
                sem0, sem1, acc):
    c = lax.axis_index("c")
    s = lax.axis_index("s")
    wid = c * NS + s

    _zero_rows(rows0, CH, D)

    for t in range(RPT // CH):
        pltpu.sync_copy(rows0, acc.at[pl.ds(s * RPT + t * CH, CH)])
    plsc.subcore_barrier()

    base = wid * CPW

    def group(g, carry):
        pltpu.sync_copy(src_hbm.at[pl.ds(base + g * GC, GC)], sidx)
        pltpu.sync_copy(dst_hbm.at[pl.ds(base + g * GC, GC)], didx)
        pltpu.async_copy(h_hbm.at[sidx.at[0]], rows0, sem0)

        def body(j, carry2):
            k = 2 * j
            pltpu.make_async_copy(h_hbm.at[sidx.at[k]], rows0, sem0).wait()
            pltpu.async_copy(h_hbm.at[sidx.at[k + 1]], rows1, sem1)
            pltpu.sync_copy(rows0, acc.at[didx.at[k]], add=True)
            pltpu.make_async_copy(h_hbm.at[sidx.at[k + 1]], rows1, sem1).wait()
            pltpu.async_copy(h_hbm.at[sidx.at[k + 2]], rows0, sem0)
            pltpu.sync_copy(rows1, acc.at[didx.at[k + 1]], add=True)
            return carry2

        lax.fori_loop(0, GC // 2 - 1, body, 0)
        kl = GC - 2
        pltpu.make_async_copy(h_hbm.at[sidx.at[kl]], rows0, sem0).wait()
        pltpu.async_copy(h_hbm.at[sidx.at[kl + 1]], rows1, sem1)
        pltpu.sync_copy(rows0, acc.at[didx.at[kl]], add=True)
        pltpu.make_async_copy(h_hbm.at[sidx.at[kl + 1]], rows1, sem1).wait()
        pltpu.sync_copy(rows1, acc.at[didx.at[kl + 1]], add=True)
        return carry

    lax.fori_loop(0, NG, group, 0)
    plsc.subcore_barrier()

    for t in range(RPT // CH):
        r = s * RPT + t * CH
        pltpu.sync_copy(acc.at[pl.ds(r, CH)], out_hbm.at[c, pl.ds(r, CH)])


def _tc_mm_body(x_ref, w_ref, u_ref):
    u_ref[...] = jnp.dot(x_ref[...], w_ref[...], preferred_element_type=jnp.float32)


def _tc_scale_body(u_ref, degs_ref, h1_ref, ns_ref, nd_ref):
    ds = (degs_ref[0, 0] + degs_ref[1, 0])[:N_NODES].reshape(N_NODES, 1)
    dd = (degs_ref[0, 1] + degs_ref[1, 1])[:N_NODES].reshape(N_NODES, 1)
    ns = lax.rsqrt(jnp.maximum(ds, 1.0))
    nd = lax.rsqrt(jnp.maximum(dd, 1.0))
    h1_ref[:N_NODES, :] = u_ref[...] * ns
    h1_ref[N_NODES:, :] = jnp.zeros((NP - N_NODES, D), jnp.float32)
    ns_ref[...] = ns
    nd_ref[...] = nd


def _tc_mid_body(p_ref, ns_ref, nd_ref, b1_ref, w2_ref, h2_ref):
    agg = p_ref[0, :N_NODES, :] + p_ref[1, :N_NODES, :]
    h = jnp.maximum(agg * nd_ref[...] + b1_ref[...][None, :], 0.0)
    h2_ref[:N_NODES, :] = (
        jnp.dot(h, w2_ref[...], preferred_element_type=jnp.float32) * ns_ref[...])
    h2_ref[N_NODES:, :] = jnp.zeros((NP - N_NODES, D), jnp.float32)


def _tc_post_body(p_ref, nd_ref, b2_ref, out_ref):
    agg = p_ref[0, :N_NODES, :] + p_ref[1, :N_NODES, :]
    out_ref[...] = agg * nd_ref[...] + b2_ref[...][None, :]


_PAD_IDX = jnp.asarray(
    N_NODES + (np.arange(EPAD - N_EDGES, dtype=np.int32) % (NP - N_NODES)),
    dtype=jnp.int32)


def kernel(in_feat, edge_index, W1, b1, W2, b2):
    src_p = jnp.concatenate([edge_index[0], _PAD_IDX]).reshape(NCH, CH)
    dst_p = jnp.concatenate([edge_index[1], _PAD_IDX]).reshape(NCH, CH)

    u1 = pl.pallas_call(
        _tc_mm_body,
        out_shape=jax.ShapeDtypeStruct((N_NODES, D), jnp.float32),
    )(in_feat, W1)

    degs = _sc_degrees(src_p, dst_p)

    h1, ns, nd = pl.pallas_call(
        _tc_scale_body,
        out_shape=[
            jax.ShapeDtypeStruct((NP, D), jnp.float32),
            jax.ShapeDtypeStruct((N_NODES, 1), jnp.float32),
            jax.ShapeDtypeStruct((N_NODES, 1), jnp.float32),
        ],
    )(u1, degs)

    p1 = _sc_msgpass(h1, src_p, dst_p)

    h2 = pl.pallas_call(
        _tc_mid_body,
        out_shape=jax.ShapeDtypeStruct((NP, D), jnp.float32),
    )(p1, ns, nd, b1, W2)

    p2 = _sc_msgpass(h2, src_p, dst_p)

    out = pl.pallas_call(
        _tc_post_body,
        out_shape=jax.ShapeDtypeStruct((N_NODES, D), jnp.float32),
    )(p2, nd, b2)
    return out


# speedup vs baseline: 1.2636x; 1.0687x over previous
import functools

import jax
import jax.numpy as jnp
import numpy as np
from jax import lax
from jax.experimental import pallas as pl
from jax.experimental.pallas import tpu as pltpu
from jax.experimental.pallas import tpu_sc as plsc

N_NODES = 10000
N_EDGES = 320000
D = 128

NC = 2
NS = 16
NW = NC * NS

CH = 128
CPW = 80
NCH = NW * CPW
EPAD = NCH * CH

NP = 10240
RPT = NP // NS
DW = 16

_mesh = plsc.VectorSubcoreMesh(core_axis_name="c", subcore_axis_name="s")


def _zero_rows(ref, nrows, width):
    groups = width // 16

    def body(i, carry):
        for j in range(groups):
            ref[i, pl.ds(j * 16, 16)] = jnp.zeros((16,), jnp.float32)
        return carry

    lax.fori_loop(0, nrows, body, 0)


def _fill_1d(ref, n, value):
    def body(i, carry):
        ref[pl.ds(i * 16, 16)] = jnp.full((16,), value, jnp.float32)
        return carry

    lax.fori_loop(0, n // 16, body, 0)


@functools.partial(
    pl.kernel,
    out_type=jax.ShapeDtypeStruct((NC, 2, NP), jnp.float32),
    mesh=_mesh,
    scratch_types=[
        pltpu.VMEM((CPW, CH), jnp.int32),
        pltpu.VMEM((CPW, CH), jnp.int32),
        pltpu.VMEM((CH,), jnp.float32),
        pltpu.VMEM((RPT,), jnp.float32),
        pltpu.SemaphoreType.DMA,
        pltpu.SemaphoreType.DMA,
        pltpu.VMEM_SHARED((NP,), jnp.float32),
        pltpu.VMEM_SHARED((NP,), jnp.float32),
    ],
)
def _sc_degrees(e_hbm, out_hbm, sidx, didx, ones_v, zb, sa, sb,
                acc_s, acc_d):
    c = lax.axis_index("c")
    s = lax.axis_index("s")
    wid = c * NS + s

    _fill_1d(zb, RPT, 0.0)
    _fill_1d(ones_v, CH, 1.0)

    pltpu.sync_copy(zb, acc_s.at[pl.ds(s * RPT, RPT)])
    pltpu.sync_copy(zb, acc_d.at[pl.ds(s * RPT, RPT)])
    plsc.subcore_barrier()

    base = wid * CPW
    pltpu.sync_copy(e_hbm.at[0, pl.ds(base, CPW)], sidx)
    pltpu.sync_copy(e_hbm.at[1, pl.ds(base, CPW)], didx)

    pltpu.async_copy(ones_v, acc_s.at[sidx.at[0]], sa, add=True)
    pltpu.async_copy(ones_v, acc_d.at[didx.at[0]], sb, add=True)

    def body(k, carry):
        pltpu.async_copy(ones_v, acc_s.at[sidx.at[k]], sa, add=True)
        pltpu.async_copy(ones_v, acc_d.at[didx.at[k]], sb, add=True)
        pltpu.make_async_copy(ones_v, acc_s.at[sidx.at[0]], sa).wait()
        pltpu.make_async_copy(ones_v, acc_d.at[didx.at[0]], sb).wait()
        return carry

    lax.fori_loop(1, CPW, body, 0)
    pltpu.make_async_copy(ones_v, acc_s.at[sidx.at[0]], sa).wait()
    pltpu.make_async_copy(ones_v, acc_d.at[didx.at[0]], sb).wait()
    plsc.subcore_barrier()

    pltpu.sync_copy(acc_s.at[pl.ds(s * RPT, RPT)], out_hbm.at[c, 0, pl.ds(s * RPT, RPT)])
    pltpu.sync_copy(acc_d.at[pl.ds(s * RPT, RPT)], out_hbm.at[c, 1, pl.ds(s * RPT, RPT)])


GC = 40
NG = CPW // GC


@functools.partial(
    pl.kernel,
    out_type=jax.ShapeDtypeStruct((NC, NP, D), jnp.float32),
    mesh=_mesh,
    scratch_types=[
        pltpu.VMEM((GC, CH), jnp.int32),
        pltpu.VMEM((GC, CH), jnp.int32),
        pltpu.VMEM((CH, D), jnp.float32),
        pltpu.VMEM((CH, D), jnp.float32),
        pltpu.SemaphoreType.DMA,
        pltpu.SemaphoreType.DMA,
        pltpu.VMEM_SHARED((NP, D), jnp.float32),
    ],
)
def _sc_msgpass(h_hbm, e_hbm, out_hbm, sidx, didx, rows0, rows1,
                sem0, sem1, acc):
    c = lax.axis_index("c")
    s = lax.axis_index("s")
    wid = c * NS + s
    base = wid * CPW

    def load_group(g):
        pltpu.sync_copy(e_hbm.at[0, pl.ds(base + g * GC, GC)], sidx)
        pltpu.sync_copy(e_hbm.at[1, pl.ds(base + g * GC, GC)], didx)
        pltpu.async_copy(h_hbm.at[sidx.at[0]], rows0, sem0)

    def run_group():
        def body(j, carry2):
            k = 2 * j
            pltpu.make_async_copy(h_hbm.at[sidx.at[k]], rows0, sem0).wait()
            pltpu.async_copy(h_hbm.at[sidx.at[k + 1]], rows1, sem1)
            pltpu.sync_copy(rows0, acc.at[didx.at[k]], add=True)
            pltpu.make_async_copy(h_hbm.at[sidx.at[k + 1]], rows1, sem1).wait()
            pltpu.async_copy(h_hbm.at[sidx.at[k + 2]], rows0, sem0)
            pltpu.sync_copy(rows1, acc.at[didx.at[k + 1]], add=True)
            return carry2

        lax.fori_loop(0, GC // 2 - 1, body, 0)
        kl = GC - 2
        pltpu.make_async_copy(h_hbm.at[sidx.at[kl]], rows0, sem0).wait()
        pltpu.async_copy(h_hbm.at[sidx.at[kl + 1]], rows1, sem1)
        pltpu.sync_copy(rows0, acc.at[didx.at[kl]], add=True)
        pltpu.make_async_copy(h_hbm.at[sidx.at[kl + 1]], rows1, sem1).wait()
        pltpu.sync_copy(rows1, acc.at[didx.at[kl + 1]], add=True)

    _zero_rows(rows1, CH, D)
    load_group(0)
    for t in range(RPT // CH):
        pltpu.sync_copy(rows1, acc.at[pl.ds(s * RPT + t * CH, CH)])
    plsc.subcore_barrier()

    run_group()
    load_group(1)
    run_group()
    plsc.subcore_barrier()

    for t in range(RPT // CH):
        r = s * RPT + t * CH
        pltpu.sync_copy(acc.at[pl.ds(r, CH)], out_hbm.at[c, pl.ds(r, CH)])


def _tc_mm_body(x_ref, w_ref, u_ref):
    u_ref[...] = jnp.dot(x_ref[...], w_ref[...], preferred_element_type=jnp.float32)


def _tc_scale_body(u_ref, degs_ref, h1_ref, ns_ref, nd_ref):
    ds = (degs_ref[0, 0] + degs_ref[1, 0])[:N_NODES].reshape(N_NODES, 1)
    dd = (degs_ref[0, 1] + degs_ref[1, 1])[:N_NODES].reshape(N_NODES, 1)
    ns = lax.rsqrt(jnp.maximum(ds, 1.0))
    nd = lax.rsqrt(jnp.maximum(dd, 1.0))
    h1_ref[:N_NODES, :] = u_ref[...] * ns
    h1_ref[N_NODES:, :] = jnp.zeros((NP - N_NODES, D), jnp.float32)
    ns_ref[...] = ns
    nd_ref[...] = nd


def _tc_mid_body(p_ref, ns_ref, nd_ref, b1_ref, w2_ref, h2_ref):
    agg = p_ref[0, :N_NODES, :] + p_ref[1, :N_NODES, :]
    h = jnp.maximum(agg * nd_ref[...] + b1_ref[...][None, :], 0.0)
    h2_ref[:N_NODES, :] = (
        jnp.dot(h, w2_ref[...], preferred_element_type=jnp.float32) * ns_ref[...])
    h2_ref[N_NODES:, :] = jnp.zeros((NP - N_NODES, D), jnp.float32)


def _tc_post_body(p_ref, nd_ref, b2_ref, out_ref):
    agg = p_ref[0, :N_NODES, :] + p_ref[1, :N_NODES, :]
    out_ref[...] = agg * nd_ref[...] + b2_ref[...][None, :]


_PAD_IDX = jnp.asarray(
    np.broadcast_to(
        N_NODES + (np.arange(EPAD - N_EDGES, dtype=np.int32) % (NP - N_NODES)),
        (2, EPAD - N_EDGES)),
    dtype=jnp.int32)


def kernel(in_feat, edge_index, W1, b1, W2, b2):
    e_p = jnp.concatenate([edge_index, _PAD_IDX], axis=1).reshape(2, NCH, CH)

    u1 = pl.pallas_call(
        _tc_mm_body,
        out_shape=jax.ShapeDtypeStruct((N_NODES, D), jnp.float32),
    )(in_feat, W1)

    degs = _sc_degrees(e_p)

    h1, ns, nd = pl.pallas_call(
        _tc_scale_body,
        out_shape=[
            jax.ShapeDtypeStruct((NP, D), jnp.float32),
            jax.ShapeDtypeStruct((N_NODES, 1), jnp.float32),
            jax.ShapeDtypeStruct((N_NODES, 1), jnp.float32),
        ],
    )(u1, degs)

    p1 = _sc_msgpass(h1, e_p)

    h2 = pl.pallas_call(
        _tc_mid_body,
        out_shape=jax.ShapeDtypeStruct((NP, D), jnp.float32),
    )(p1, ns, nd, b1, W2)

    p2 = _sc_msgpass(h2, e_p)

    out = pl.pallas_call(
        _tc_post_body,
        out_shape=jax.ShapeDtypeStruct((N_NODES, D), jnp.float32),
    )(p2, nd, b2)
    return out


# speedup vs baseline: 1.2666x; 1.0024x over previous
import functools

import jax
import jax.numpy as jnp
import numpy as np
from jax import lax
from jax.experimental import pallas as pl
from jax.experimental.pallas import tpu as pltpu
from jax.experimental.pallas import tpu_sc as plsc

N_NODES = 10000
N_EDGES = 320000
D = 128

NC = 2
NS = 16
NW = NC * NS

CH = 128
CPW = 80
NCH = NW * CPW
EPAD = NCH * CH

NP = 10240
RPT = NP // NS
DW = 16

_mesh = plsc.VectorSubcoreMesh(core_axis_name="c", subcore_axis_name="s")


def _zero_rows(ref, nrows, width):
    groups = width // 16

    def body(i, carry):
        for j in range(groups):
            ref[i, pl.ds(j * 16, 16)] = jnp.zeros((16,), jnp.float32)
        return carry

    lax.fori_loop(0, nrows, body, 0)


def _fill_1d(ref, n, value):
    def body(i, carry):
        ref[pl.ds(i * 16, 16)] = jnp.full((16,), value, jnp.float32)
        return carry

    lax.fori_loop(0, n // 16, body, 0)


@functools.partial(
    pl.kernel,
    out_type=jax.ShapeDtypeStruct((NC, 2, NP), jnp.float32),
    mesh=_mesh,
    scratch_types=[
        pltpu.VMEM((CPW, CH), jnp.int32),
        pltpu.VMEM((CPW, CH), jnp.int32),
        pltpu.VMEM((CH,), jnp.float32),
        pltpu.VMEM((RPT,), jnp.float32),
        pltpu.SemaphoreType.DMA,
        pltpu.SemaphoreType.DMA,
        pltpu.VMEM_SHARED((NP,), jnp.float32),
        pltpu.VMEM_SHARED((NP,), jnp.float32),
    ],
)
def _sc_degrees(e_hbm, out_hbm, sidx, didx, ones_v, zb, sa, sb,
                acc_s, acc_d):
    c = lax.axis_index("c")
    s = lax.axis_index("s")
    wid = c * NS + s

    _fill_1d(zb, RPT, 0.0)
    _fill_1d(ones_v, CH, 1.0)

    pltpu.sync_copy(zb, acc_s.at[pl.ds(s * RPT, RPT)])
    pltpu.sync_copy(zb, acc_d.at[pl.ds(s * RPT, RPT)])
    plsc.subcore_barrier()

    base = wid * CPW
    pltpu.sync_copy(e_hbm.at[0, pl.ds(base, CPW)], sidx)
    pltpu.sync_copy(e_hbm.at[1, pl.ds(base, CPW)], didx)

    for k0 in range(2):
        pltpu.async_copy(ones_v, acc_s.at[sidx.at[k0]], sa, add=True)
        pltpu.async_copy(ones_v, acc_d.at[didx.at[k0]], sb, add=True)

    def body(k, carry):
        pltpu.async_copy(ones_v, acc_s.at[sidx.at[k]], sa, add=True)
        pltpu.async_copy(ones_v, acc_d.at[didx.at[k]], sb, add=True)
        pltpu.make_async_copy(ones_v, acc_s.at[sidx.at[0]], sa).wait()
        pltpu.make_async_copy(ones_v, acc_d.at[didx.at[0]], sb).wait()
        return carry

    lax.fori_loop(2, CPW, body, 0)
    for _ in range(2):
        pltpu.make_async_copy(ones_v, acc_s.at[sidx.at[0]], sa).wait()
        pltpu.make_async_copy(ones_v, acc_d.at[didx.at[0]], sb).wait()
    plsc.subcore_barrier()

    pltpu.sync_copy(acc_s.at[pl.ds(s * RPT, RPT)], out_hbm.at[c, 0, pl.ds(s * RPT, RPT)])
    pltpu.sync_copy(acc_d.at[pl.ds(s * RPT, RPT)], out_hbm.at[c, 1, pl.ds(s * RPT, RPT)])


GC = 40
NG = CPW // GC


@functools.partial(
    pl.kernel,
    out_type=jax.ShapeDtypeStruct((NC, NP, D), jnp.float32),
    mesh=_mesh,
    scratch_types=[
        pltpu.VMEM((GC, CH), jnp.int32),
        pltpu.VMEM((GC, CH), jnp.int32),
        pltpu.VMEM((CH, D), jnp.float32),
        pltpu.VMEM((CH, D), jnp.float32),
        pltpu.SemaphoreType.DMA,
        pltpu.SemaphoreType.DMA,
        pltpu.VMEM_SHARED((NP, D), jnp.float32),
    ],
)
def _sc_msgpass(h_hbm, e_hbm, out_hbm, sidx, didx, rows0, rows1,
                sem0, sem1, acc):
    c = lax.axis_index("c")
    s = lax.axis_index("s")
    wid = c * NS + s
    base = wid * CPW

    def load_group(g):
        pltpu.sync_copy(e_hbm.at[0, pl.ds(base + g * GC, GC)], sidx)
        pltpu.sync_copy(e_hbm.at[1, pl.ds(base + g * GC, GC)], didx)
        pltpu.async_copy(h_hbm.at[sidx.at[0]], rows0, sem0)

    def run_group():
        def body(j, carry2):
            k = 2 * j
            pltpu.make_async_copy(h_hbm.at[sidx.at[k]], rows0, sem0).wait()
            pltpu.async_copy(h_hbm.at[sidx.at[k + 1]], rows1, sem1)
            pltpu.sync_copy(rows0, acc.at[didx.at[k]], add=True)
            pltpu.make_async_copy(h_hbm.at[sidx.at[k + 1]], rows1, sem1).wait()
            pltpu.async_copy(h_hbm.at[sidx.at[k + 2]], rows0, sem0)
            pltpu.sync_copy(rows1, acc.at[didx.at[k + 1]], add=True)
            return carry2

        lax.fori_loop(0, GC // 2 - 1, body, 0)
        kl = GC - 2
        pltpu.make_async_copy(h_hbm.at[sidx.at[kl]], rows0, sem0).wait()
        pltpu.async_copy(h_hbm.at[sidx.at[kl + 1]], rows1, sem1)
        pltpu.sync_copy(rows0, acc.at[didx.at[kl]], add=True)
        pltpu.make_async_copy(h_hbm.at[sidx.at[kl + 1]], rows1, sem1).wait()
        pltpu.sync_copy(rows1, acc.at[didx.at[kl + 1]], add=True)

    _zero_rows(rows1, CH, D)
    load_group(0)
    for t in range(RPT // CH):
        pltpu.sync_copy(rows1, acc.at[pl.ds(s * RPT + t * CH, CH)])
    plsc.subcore_barrier()

    run_group()
    load_group(1)
    run_group()
    plsc.subcore_barrier()

    for t in range(RPT // CH):
        r = s * RPT + t * CH
        pltpu.sync_copy(acc.at[pl.ds(r, CH)], out_hbm.at[c, pl.ds(r, CH)])


def _tc_mm_body(x_ref, w_ref, u_ref):
    u_ref[...] = jnp.dot(x_ref[...], w_ref[...], preferred_element_type=jnp.float32)


def _tc_scale_body(u_ref, degs_ref, h1_ref, ns_ref, nd_ref):
    ds = (degs_ref[0, 0] + degs_ref[1, 0])[:N_NODES].reshape(N_NODES, 1)
    dd = (degs_ref[0, 1] + degs_ref[1, 1])[:N_NODES].reshape(N_NODES, 1)
    ns = lax.rsqrt(jnp.maximum(ds, 1.0))
    nd = lax.rsqrt(jnp.maximum(dd, 1.0))
    h1_ref[:N_NODES, :] = u_ref[...] * ns
    h1_ref[N_NODES:, :] = jnp.zeros((NP - N_NODES, D), jnp.float32)
    ns_ref[...] = ns
    nd_ref[...] = nd


def _tc_mid_body(p_ref, ns_ref, nd_ref, b1_ref, w2_ref, h2_ref):
    agg = p_ref[0, :N_NODES, :] + p_ref[1, :N_NODES, :]
    h = jnp.maximum(agg * nd_ref[...] + b1_ref[...][None, :], 0.0)
    h2_ref[:N_NODES, :] = (
        jnp.dot(h, w2_ref[...], preferred_element_type=jnp.float32) * ns_ref[...])
    h2_ref[N_NODES:, :] = jnp.zeros((NP - N_NODES, D), jnp.float32)


def _tc_post_body(p_ref, nd_ref, b2_ref, out_ref):
    agg = p_ref[0, :N_NODES, :] + p_ref[1, :N_NODES, :]
    out_ref[...] = agg * nd_ref[...] + b2_ref[...][None, :]


_PAD_IDX = jnp.asarray(
    np.broadcast_to(
        N_NODES + (np.arange(EPAD - N_EDGES, dtype=np.int32) % (NP - N_NODES)),
        (2, EPAD - N_EDGES)),
    dtype=jnp.int32)


def kernel(in_feat, edge_index, W1, b1, W2, b2):
    e_p = jnp.concatenate([edge_index, _PAD_IDX], axis=1).reshape(2, NCH, CH)

    u1 = pl.pallas_call(
        _tc_mm_body,
        out_shape=jax.ShapeDtypeStruct((N_NODES, D), jnp.float32),
    )(in_feat, W1)

    degs = _sc_degrees(e_p)

    h1, ns, nd = pl.pallas_call(
        _tc_scale_body,
        out_shape=[
            jax.ShapeDtypeStruct((NP, D), jnp.float32),
            jax.ShapeDtypeStruct((N_NODES, 1), jnp.float32),
            jax.ShapeDtypeStruct((N_NODES, 1), jnp.float32),
        ],
    )(u1, degs)

    p1 = _sc_msgpass(h1, e_p)

    h2 = pl.pallas_call(
        _tc_mid_body,
        out_shape=jax.ShapeDtypeStruct((NP, D), jnp.float32),
    )(p1, ns, nd, b1, W2)

    p2 = _sc_msgpass(h2, e_p)

    out = pl.pallas_call(
        _tc_post_body,
        out_shape=jax.ShapeDtypeStruct((N_NODES, D), jnp.float32),
    )(p2, nd, b2)
    return out
